# baseline (speedup 1.0000x reference)
import functools

import jax
import jax.numpy as jnp
from jax import lax
from jax.experimental import pallas as pl
from jax.experimental.pallas import tpu as pltpu
from jax.experimental.pallas import tpu_sc as plsc

_DIM = 128
_B = 1024
_S = 200
_Q = 512
_NTAB = 1000

_NC = 2
_NS = 16
_NW = _NC * _NS

_TOTAL = _B * _S
_ROWS_PER_W = _TOTAL // _NW
_IDXR_PER_W = _B // _NW
_C0 = 128
_C1 = _S - _C0


def _make_sc_gather():
  mesh = plsc.VectorSubcoreMesh(core_axis_name="c", subcore_axis_name="s")

  @functools.partial(
      pl.kernel,
      mesh=mesh,
      out_type=jax.ShapeDtypeStruct((_TOTAL, _DIM), jnp.float32),
      scratch_types=[
          pltpu.VMEM((_IDXR_PER_W, _S), jnp.int32),
          pltpu.VMEM((_C0, _DIM), jnp.float32),
          pltpu.VMEM((_C0, _DIM), jnp.float32),
          pltpu.VMEM_SHARED((_NTAB, _DIM), jnp.float32),
          pltpu.SemaphoreType.DMA,
          pltpu.SemaphoreType.DMA,
      ],
  )
  def sc_gather(table_hbm, idx_hbm, out_hbm, idx_v, rows_a, rows_b,
                table_sp, sem_a, sem_b):
    sid = lax.axis_index("s")
    wid = sid * _NC + lax.axis_index("c")
    out_base = wid * _ROWS_PER_W

    @pl.when(sid == 0)
    def _():
      pltpu.sync_copy(table_hbm, table_sp)

    pltpu.sync_copy(idx_hbm.at[pl.ds(wid * _IDXR_PER_W, _IDXR_PER_W)], idx_v)
    plsc.subcore_barrier()

    def start_gather(r, half, buf, sem):
      if half == 0:
        pltpu.async_copy(table_sp.at[idx_v.at[r, pl.ds(0, _C0)]], buf, sem)
      else:
        pltpu.async_copy(
            table_sp.at[idx_v.at[r, pl.ds(_C0, _C1)]],
            buf.at[pl.ds(0, _C1)], sem,
        )

    def drain(half, buf, sem):
      n = _C0 if half == 0 else _C1
      pltpu.make_async_copy(
          table_hbm.at[pl.ds(0, n)], buf.at[pl.ds(0, n)], sem
      ).wait()

    def write_out(r, half, buf):
      off = out_base + r * _S + (0 if half == 0 else _C0)
      n = _C0 if half == 0 else _C1
      pltpu.sync_copy(buf.at[pl.ds(0, n)], out_hbm.at[pl.ds(off, n)])

    start_gather(0, 0, rows_a, sem_a)

    def row_body(r, carry):
      start_gather(r, 1, rows_b, sem_b)
      drain(0, rows_a, sem_a)
      write_out(r, 0, rows_a)

      @pl.when(r < _IDXR_PER_W - 1)
      def _():
        start_gather(r + 1, 0, rows_a, sem_a)

      drain(1, rows_b, sem_b)
      write_out(r, 1, rows_b)
      return carry

    lax.fori_loop(0, _IDXR_PER_W, row_body, 0)

  return sc_gather


_sc_gather = _make_sc_gather()

_BCAST_ROWS = 8192
_BCAST_REPS = (_B * _Q) // _BCAST_ROWS
_BCAST_NSEM = 4


def _bcast_body(mask_ref, out_hbm, stage_ref, sems):
  stage_ref[...] = jnp.broadcast_to(mask_ref[...], stage_ref.shape)
  seed = out_hbm.at[pl.ds(0, _BCAST_ROWS)]
  pltpu.make_async_copy(stage_ref, seed, sems.at[0]).start()
  pltpu.make_async_copy(stage_ref, seed, sems.at[0]).wait()
  for i in range(1, _BCAST_REPS):
    pltpu.make_async_copy(
        seed, out_hbm.at[pl.ds(i * _BCAST_ROWS, _BCAST_ROWS)],
        sems.at[i % _BCAST_NSEM],
    ).start()
  for i in range(1, _BCAST_REPS):
    pltpu.make_async_copy(
        seed, out_hbm.at[pl.ds(i * _BCAST_ROWS, _BCAST_ROWS)],
        sems.at[i % _BCAST_NSEM],
    ).wait()


_bcast = pl.pallas_call(
    _bcast_body,
    in_specs=[pl.BlockSpec(memory_space=pltpu.VMEM)],
    out_specs=pl.BlockSpec(memory_space=pl.ANY),
    out_shape=jax.ShapeDtypeStruct((_B * _Q, _DIM), jnp.float32),
    scratch_shapes=[
        pltpu.VMEM((_BCAST_ROWS, _DIM), jnp.float32),
        pltpu.SemaphoreType.DMA((_BCAST_NSEM,)),
    ],
)


def kernel(y_support, n_obs_query, y_embedding, y_mask):
  del n_obs_query
  table_shift = lax.slice(y_embedding, (1, 0), (_NTAB + 1, _DIM))
  emb = _sc_gather(table_shift, y_support)
  y_query = _bcast(y_mask)
  return (emb.reshape(_B, _S, _DIM), y_query.reshape(_B, _Q, _DIM))


# speedup vs baseline: 61.4640x; 61.4640x over previous
import functools

import jax
import jax.numpy as jnp
from jax import lax
from jax.experimental import pallas as pl
from jax.experimental.pallas import tpu as pltpu
from jax.experimental.pallas import tpu_sc as plsc

_DIM = 128
_B = 1024
_S = 200
_Q = 512
_NTAB = 1000

_NC = 2
_NS = 16
_NW = _NC * _NS

_TOTAL = _B * _S
_ROWS_PER_W = _TOTAL // _NW
_IDXR_PER_W = _B // _NW
_C0 = 128
_C1 = _S - _C0


def _make_sc_gather():
  mesh = plsc.VectorSubcoreMesh(core_axis_name="c", subcore_axis_name="s")

  @functools.partial(
      pl.kernel,
      mesh=mesh,
      out_type=jax.ShapeDtypeStruct((_TOTAL, _DIM), jnp.float32),
      scratch_types=[
          pltpu.VMEM((_IDXR_PER_W, _S), jnp.int32),
          pltpu.VMEM((_C0, _DIM), jnp.float32),
          pltpu.VMEM((_C0, _DIM), jnp.float32),
          pltpu.VMEM((8, _DIM), jnp.float32),
          pltpu.VMEM_SHARED((_NTAB, _DIM), jnp.float32),
          pltpu.SemaphoreType.DMA,
          pltpu.SemaphoreType.DMA,
      ],
  )
  def sc_gather(table_hbm, idx_hbm, out_hbm, idx_v, rows_a, rows_b,
                head_v, table_sp, sem_a, sem_b):
    sid = lax.axis_index("s")
    wid = sid * _NC + lax.axis_index("c")
    out_base = wid * _ROWS_PER_W

    @pl.when(sid == 0)
    def _():
      pltpu.sync_copy(table_hbm.at[pl.ds(8, _NTAB - 7)],
                      table_sp.at[pl.ds(7, _NTAB - 7)])
      pltpu.sync_copy(table_hbm.at[pl.ds(0, 8)], head_v)
      pltpu.sync_copy(head_v.at[pl.ds(1, 7)], table_sp.at[pl.ds(0, 7)])

    pltpu.sync_copy(idx_hbm.at[pl.ds(wid * _IDXR_PER_W, _IDXR_PER_W)], idx_v)
    plsc.subcore_barrier()

    def start_gather(r, half, buf, sem):
      if half == 0:
        pltpu.async_copy(table_sp.at[idx_v.at[r, pl.ds(0, _C0)]], buf, sem)
      else:
        pltpu.async_copy(
            table_sp.at[idx_v.at[r, pl.ds(_C0, _C1)]],
            buf.at[pl.ds(0, _C1)], sem,
        )

    def drain(half, buf, sem):
      n = _C0 if half == 0 else _C1
      pltpu.make_async_copy(
          table_hbm.at[pl.ds(0, n)], buf.at[pl.ds(0, n)], sem
      ).wait()

    def write_out(r, half, buf):
      off = out_base + r * _S + (0 if half == 0 else _C0)
      n = _C0 if half == 0 else _C1
      pltpu.sync_copy(buf.at[pl.ds(0, n)], out_hbm.at[pl.ds(off, n)])

    start_gather(0, 0, rows_a, sem_a)

    def row_body(r, carry):
      start_gather(r, 1, rows_b, sem_b)
      drain(0, rows_a, sem_a)
      write_out(r, 0, rows_a)

      @pl.when(r < _IDXR_PER_W - 1)
      def _():
        start_gather(r + 1, 0, rows_a, sem_a)

      drain(1, rows_b, sem_b)
      write_out(r, 1, rows_b)
      return carry

    lax.fori_loop(0, _IDXR_PER_W, row_body, 0)

  return sc_gather


_sc_gather = _make_sc_gather()

_BCAST_ROWS = 8192
_BCAST_GRID = (_B * _Q) // _BCAST_ROWS


def _bcast_body(mask_ref, out_ref):
  out_ref[...] = jnp.broadcast_to(mask_ref[...], out_ref.shape)


_bcast = pl.pallas_call(
    _bcast_body,
    grid=(_BCAST_GRID,),
    in_specs=[pl.BlockSpec((1, _DIM), lambda i: (0, 0))],
    out_specs=pl.BlockSpec((_BCAST_ROWS, _DIM), lambda i: (i, 0)),
    out_shape=jax.ShapeDtypeStruct((_B * _Q, _DIM), jnp.float32),
)


def kernel(y_support, n_obs_query, y_embedding, y_mask):
  del n_obs_query
  emb = _sc_gather(y_embedding, y_support)
  y_query = _bcast(y_mask)
  return (emb.reshape(_B, _S, _DIM), y_query.reshape(_B, _Q, _DIM))
